# baseline (speedup 1.0000x reference)
import functools

import jax
import jax.numpy as jnp
from jax import lax
from jax.experimental import pallas as pl
from jax.experimental.pallas import tpu as pltpu
from jax.experimental.pallas import tpu_sc as plsc

N = 10000
E = 320000
D = 128
NC = 2
NS = 16
NW = NC * NS
EPW = E // NW
C = 80
NCHUNK = EPW // C
GC = 25
LANES = 16
FEAT_GROUPS = D // LANES


def _mm_body(x_ref, w0_ref, w1_ref, o0_ref, o1_ref):
    xb = x_ref[...]
    o0_ref[...] = jnp.dot(xb, w0_ref[...], preferred_element_type=jnp.float32)
    o1_ref[...] = jnp.dot(xb, w1_ref[...], preferred_element_type=jnp.float32)


def _matmul(x, W0, W1):
    blk = 2000
    return pl.pallas_call(
        _mm_body,
        grid=(N // blk,),
        in_specs=[
            pl.BlockSpec((blk, D), lambda i: (i, 0)),
            pl.BlockSpec((D, D), lambda i: (0, 0)),
            pl.BlockSpec((D, D), lambda i: (0, 0)),
        ],
        out_specs=[
            pl.BlockSpec((blk, D), lambda i: (i, 0)),
            pl.BlockSpec((blk, D), lambda i: (i, 0)),
        ],
        out_shape=[
            jax.ShapeDtypeStruct((N, D), jnp.float32),
            jax.ShapeDtypeStruct((N, D), jnp.float32),
        ],
    )(x, W0, W1)


SPL = 48


def _sc_body(pre0, pre1, ra0, rb0, c0, v0, ra1, rb1, c1, v1, out,
             rowa_b, rowb_b, col_b, val_b, buf0, buf1, buf2, acc,
             sg0, sg1, sg2, ss0, ss1, ss2):
    bufs = (buf0, buf1, buf2)
    sgs = (sg0, sg1, sg2)
    sss = (ss0, ss1, ss2)
    cid = lax.axis_index("c")
    sid = lax.axis_index("s")
    wid = sid * NC + cid

    zeros16 = jnp.zeros((LANES,), jnp.float32)
    n_acc_chunks = N // C

    def _zero_row(i, carry):
        for j in range(FEAT_GROUPS):
            buf0[i, pl.ds(j * LANES, LANES)] = zeros16
        return carry

    lax.fori_loop(0, C, _zero_row, 0)
    for k in range(pl.cdiv(n_acc_chunks, NS)):
        m = sid + NS * k

        @pl.when(m < n_acc_chunks)
        def _():
            pltpu.sync_copy(buf0, acc.at[pl.ds(m * C, C)])

    plsc.subcore_barrier()

    def _scale_range(buf, i, g0, g1):
        def _scale(gg, c2):
            e0 = gg * LANES
            v16 = val_b[i, pl.ds(e0, LANES)]
            for k in range(LANES):
                val = v16[k]
                for j in range(FEAT_GROUPS):
                    sl = pl.ds(j * LANES, LANES)
                    buf[e0 + k, sl] = buf[e0 + k, sl] * val
            return c2

        lax.fori_loop(g0, g1, _scale, 0)

    def _scat(buf, i, half, sem, is_wait):
        lo, nr, idx = ((0, SPL, rowa_b) if half == 0
                       else (SPL, C - SPL, rowb_b))
        if is_wait:
            pltpu.make_async_copy(
                buf.at[pl.ds(lo, nr)], acc.at[idx.at[i]], sem).wait()
        else:
            pltpu.async_copy(
                buf.at[pl.ds(lo, nr)], acc.at[idx.at[i]], sem, add=True)

    for pre, ra_hbm, rb_hbm, c_hbm, v_hbm in (
            (pre0, ra0, rb0, c0, v0), (pre1, ra1, rb1, c1, v1)):

        def _group(g, carry):
            pltpu.sync_copy(ra_hbm.at[wid, g], rowa_b)
            pltpu.sync_copy(rb_hbm.at[wid, g], rowb_b)
            pltpu.sync_copy(c_hbm.at[wid, g], col_b)
            pltpu.sync_copy(v_hbm.at[wid, g], val_b)
            pltpu.async_copy(pre.at[col_b.at[0]], buf0, sg0)
            pltpu.async_copy(pre.at[col_b.at[1]], buf1, sg1)

            def _triple(t, c2):
                for leg in range(3):
                    c = 3 * t + leg
                    bc = bufs[leg]
                    bn = bufs[(leg + 2) % 3]
                    sn = sss[(leg + 2) % 3]
                    gn = sgs[(leg + 2) % 3]
                    pltpu.make_async_copy(pre.at[col_b.at[c]], bc, sgs[leg]).wait()
                    _scale_range(bc, c, 0, SPL // LANES)
                    _scat(bc, c, 0, sss[leg], False)
                    _scale_range(bc, c, SPL // LANES, C // LANES)
                    _scat(bc, c, 1, sss[leg], False)

                    def _drain(c=c, bn=bn, sn=sn):
                        _scat(bn, c - 1, 0, sn, True)
                        _scat(bn, c - 1, 1, sn, True)

                    def _prefetch(c=c, bn=bn, gn=gn):
                        pltpu.async_copy(pre.at[col_b.at[c + 2]], bn, gn)

                    if leg == 0:
                        @pl.when(t > 0)
                        def _():
                            _drain()

                        _prefetch()
                    elif leg == 1:
                        _drain()
                        _prefetch()
                    else:
                        _drain()

                        @pl.when(3 * t + 4 < GC)
                        def _():
                            _prefetch()
                return c2

            lax.fori_loop(0, (GC - 1) // 3, _triple, 0)

            last = GC - 1
            pltpu.make_async_copy(pre.at[col_b.at[last]], buf0, sg0).wait()
            _scat(buf2, last - 1, 0, ss2, True)
            _scat(buf2, last - 1, 1, ss2, True)
            _scale_range(buf0, last, 0, C // LANES)
            pltpu.sync_copy(buf0.at[pl.ds(0, SPL)],
                            acc.at[rowa_b.at[last]], add=True)
            pltpu.sync_copy(buf0.at[pl.ds(SPL, C - SPL)],
                            acc.at[rowb_b.at[last]], add=True)
            return carry

        lax.fori_loop(0, NCHUNK // GC, _group, 0)

    plsc.subcore_barrier()

    for k in range(pl.cdiv(n_acc_chunks, NS)):
        m = sid + NS * k

        @pl.when(m < n_acc_chunks)
        def _():
            pltpu.sync_copy(acc.at[pl.ds(m * C, C)], buf0)
            pltpu.sync_copy(buf0, out.at[cid, pl.ds(m * C, C)])


_sc_agg = functools.partial(
    pl.kernel,
    mesh=plsc.VectorSubcoreMesh(core_axis_name="c", subcore_axis_name="s"),
    out_type=jax.ShapeDtypeStruct((NC, N, D), jnp.float32),
    scratch_types=[
        pltpu.VMEM((GC, SPL), jnp.int32),
        pltpu.VMEM((GC, C - SPL), jnp.int32),
        pltpu.VMEM((GC, C), jnp.int32),
        pltpu.VMEM((GC, C), jnp.float32),
        pltpu.VMEM((C, D), jnp.float32),
        pltpu.VMEM((C, D), jnp.float32),
        pltpu.VMEM((C, D), jnp.float32),
        pltpu.VMEM_SHARED((N, D), jnp.float32),
        pltpu.SemaphoreType.DMA,
        pltpu.SemaphoreType.DMA,
        pltpu.SemaphoreType.DMA,
        pltpu.SemaphoreType.DMA,
        pltpu.SemaphoreType.DMA,
        pltpu.SemaphoreType.DMA,
    ],
)(_sc_body)


def _fin_body(p_ref, b_ref, o_ref):
    s = p_ref[0] + p_ref[1] + b_ref[...]
    o_ref[...] = jnp.maximum(s, 0.0)


def _finalize(partials, b2d):
    blk = 2000
    return pl.pallas_call(
        _fin_body,
        grid=(N // blk,),
        in_specs=[
            pl.BlockSpec((NC, blk, D), lambda i: (0, i, 0)),
            pl.BlockSpec((1, D), lambda i: (0, 0)),
        ],
        out_specs=pl.BlockSpec((blk, D), lambda i: (i, 0)),
        out_shape=jax.ShapeDtypeStruct((N, D), jnp.float32),
    )(partials, b2d)


def kernel(x, support0_edge_index, support0_values,
           support1_edge_index, support1_values, W0, W1, b):
    pre0, pre1 = _matmul(x, W0, W1)
    ng = NCHUNK // GC
    r0 = support0_edge_index[0].reshape(NW, ng, GC, C)
    c0 = support0_edge_index[1].reshape(NW, ng, GC, C)
    v0 = support0_values.reshape(NW, ng, GC, C)
    r1 = support1_edge_index[0].reshape(NW, ng, GC, C)
    c1 = support1_edge_index[1].reshape(NW, ng, GC, C)
    v1 = support1_values.reshape(NW, ng, GC, C)
    partials = _sc_agg(pre0, pre1,
                       r0[..., :SPL], r0[..., SPL:], c0, v0,
                       r1[..., :SPL], r1[..., SPL:], c1, v1)
    return _finalize(partials, b.reshape(1, D))


# speedup vs baseline: 1.0553x; 1.0553x over previous
import functools

import jax
import jax.numpy as jnp
from jax import lax
from jax.experimental import pallas as pl
from jax.experimental.pallas import tpu as pltpu
from jax.experimental.pallas import tpu_sc as plsc

N = 10000
E = 320000
D = 128
NC = 2
NS = 16
NW = NC * NS
EPW = E // NW
C = 80
NCHUNK = EPW // C
GC = 25
LANES = 16
FEAT_GROUPS = D // LANES


def _mm_body(x_ref, w0_ref, w1_ref, o0_ref, o1_ref):
    xb = x_ref[...]
    o0_ref[...] = jnp.dot(xb, w0_ref[...], preferred_element_type=jnp.float32)
    o1_ref[...] = jnp.dot(xb, w1_ref[...], preferred_element_type=jnp.float32)


def _matmul(x, W0, W1):
    blk = 2000
    return pl.pallas_call(
        _mm_body,
        grid=(N // blk,),
        in_specs=[
            pl.BlockSpec((blk, D), lambda i: (i, 0)),
            pl.BlockSpec((D, D), lambda i: (0, 0)),
            pl.BlockSpec((D, D), lambda i: (0, 0)),
        ],
        out_specs=[
            pl.BlockSpec((blk, D), lambda i: (i, 0)),
            pl.BlockSpec((blk, D), lambda i: (i, 0)),
        ],
        out_shape=[
            jax.ShapeDtypeStruct((N, D), jnp.float32),
            jax.ShapeDtypeStruct((N, D), jnp.float32),
        ],
    )(x, W0, W1)


def _sc_body(pre0, pre1, r0, c0, v0, r1, c1, v1, out,
             row_b, col_b, val_b, buf0, buf1, buf2, acc,
             sg0, sg1, sg2, ss0, ss1, ss2):
    bufs = (buf0, buf1, buf2)
    sgs = (sg0, sg1, sg2)
    sss = (ss0, ss1, ss2)
    cid = lax.axis_index("c")
    sid = lax.axis_index("s")
    wid = sid * NC + cid

    zeros16 = jnp.zeros((LANES,), jnp.float32)
    n_acc_chunks = N // C

    def _zero_row(i, carry):
        for j in range(FEAT_GROUPS):
            buf0[i, pl.ds(j * LANES, LANES)] = zeros16
        return carry

    lax.fori_loop(0, C, _zero_row, 0)
    for k in range(pl.cdiv(n_acc_chunks, NS)):
        m = sid + NS * k

        @pl.when(m < n_acc_chunks)
        def _():
            pltpu.sync_copy(buf0, acc.at[pl.ds(m * C, C)])

    plsc.subcore_barrier()

    def _scale_buf(buf, i):
        def _scale(gg, c2):
            e0 = gg * LANES
            v16 = val_b[i, pl.ds(e0, LANES)]
            for k in range(LANES):
                val = v16[k]
                for j in range(FEAT_GROUPS):
                    sl = pl.ds(j * LANES, LANES)
                    buf[e0 + k, sl] = buf[e0 + k, sl] * val
            return c2

        lax.fori_loop(0, C // LANES, _scale, 0)

    for pre, r_hbm, c_hbm, v_hbm in ((pre0, r0, c0, v0), (pre1, r1, c1, v1)):

        def _group(g, carry):
            pltpu.sync_copy(r_hbm.at[wid, g], row_b)
            pltpu.sync_copy(c_hbm.at[wid, g], col_b)
            pltpu.sync_copy(v_hbm.at[wid, g], val_b)
            pltpu.async_copy(pre.at[col_b.at[0]], buf0, sg0)
            pltpu.async_copy(pre.at[col_b.at[1]], buf1, sg1)

            def _triple(t, c2):
                for leg in range(3):
                    c = 3 * t + leg
                    bc = bufs[leg]
                    bn = bufs[(leg + 2) % 3]
                    sn = sss[(leg + 2) % 3]
                    gn = sgs[(leg + 2) % 3]
                    pltpu.make_async_copy(pre.at[col_b.at[c]], bc, sgs[leg]).wait()
                    _scale_buf(bc, c)
                    pltpu.async_copy(bc, acc.at[row_b.at[c]], sss[leg], add=True)

                    def _drain(c=c, bn=bn, sn=sn):
                        pltpu.make_async_copy(bn, acc.at[row_b.at[c - 1]], sn).wait()

                    def _prefetch(c=c, bn=bn, gn=gn):
                        pltpu.async_copy(pre.at[col_b.at[c + 2]], bn, gn)

                    if leg == 0:
                        @pl.when(t > 0)
                        def _():
                            _drain()

                        _prefetch()
                    elif leg == 1:
                        _drain()
                        _prefetch()
                    else:
                        _drain()

                        @pl.when(3 * t + 4 < GC)
                        def _():
                            _prefetch()
                return c2

            lax.fori_loop(0, (GC - 1) // 3, _triple, 0)

            last = GC - 1
            pltpu.make_async_copy(pre.at[col_b.at[last]], buf0, sg0).wait()
            pltpu.make_async_copy(buf2, acc.at[row_b.at[last - 1]], ss2).wait()
            _scale_buf(buf0, last)
            pltpu.sync_copy(buf0, acc.at[row_b.at[last]], add=True)
            return carry

        lax.fori_loop(0, NCHUNK // GC, _group, 0)

    plsc.subcore_barrier()

    for k in range(pl.cdiv(n_acc_chunks, NS)):
        m = sid + NS * k

        @pl.when(m < n_acc_chunks)
        def _():
            pltpu.sync_copy(acc.at[pl.ds(m * C, C)], buf0)
            pltpu.sync_copy(buf0, out.at[cid, pl.ds(m * C, C)])


_sc_agg = functools.partial(
    pl.kernel,
    mesh=plsc.VectorSubcoreMesh(core_axis_name="c", subcore_axis_name="s"),
    out_type=jax.ShapeDtypeStruct((NC, N, D), jnp.float32),
    scratch_types=[
        pltpu.VMEM((GC, C), jnp.int32),
        pltpu.VMEM((GC, C), jnp.int32),
        pltpu.VMEM((GC, C), jnp.float32),
        pltpu.VMEM((C, D), jnp.float32),
        pltpu.VMEM((C, D), jnp.float32),
        pltpu.VMEM((C, D), jnp.float32),
        pltpu.VMEM_SHARED((N, D), jnp.float32),
        pltpu.SemaphoreType.DMA,
        pltpu.SemaphoreType.DMA,
        pltpu.SemaphoreType.DMA,
        pltpu.SemaphoreType.DMA,
        pltpu.SemaphoreType.DMA,
        pltpu.SemaphoreType.DMA,
    ],
)(_sc_body)


def _fin_body(p_ref, b_ref, o_ref):
    s = p_ref[0] + p_ref[1] + b_ref[...]
    o_ref[...] = jnp.maximum(s, 0.0)


def _finalize(partials, b2d):
    blk = 2000
    return pl.pallas_call(
        _fin_body,
        grid=(N // blk,),
        in_specs=[
            pl.BlockSpec((NC, blk, D), lambda i: (0, i, 0)),
            pl.BlockSpec((1, D), lambda i: (0, 0)),
        ],
        out_specs=pl.BlockSpec((blk, D), lambda i: (i, 0)),
        out_shape=jax.ShapeDtypeStruct((N, D), jnp.float32),
    )(partials, b2d)


def kernel(x, support0_edge_index, support0_values,
           support1_edge_index, support1_values, W0, W1, b):
    pre0, pre1 = _matmul(x, W0, W1)
    ng = NCHUNK // GC
    r0 = support0_edge_index[0].reshape(NW, ng, GC, C)
    c0 = support0_edge_index[1].reshape(NW, ng, GC, C)
    v0 = support0_values.reshape(NW, ng, GC, C)
    r1 = support1_edge_index[0].reshape(NW, ng, GC, C)
    c1 = support1_edge_index[1].reshape(NW, ng, GC, C)
    v1 = support1_values.reshape(NW, ng, GC, C)
    partials = _sc_agg(pre0, pre1, r0, c0, v0, r1, c1, v1)
    return _finalize(partials, b.reshape(1, D))
